# Initial kernel scaffold; baseline (speedup 1.0000x reference)
import jax
import jax.numpy as jnp
from jax.experimental import pallas as pl


def kernel(latent, codebook):
    raise NotImplementedError("write your pallas kernel here")


# speedup vs baseline: 1.4507x; 1.4507x over previous
import jax
import jax.numpy as jnp
from jax.experimental import pallas as pl
from jax.experimental.pallas import tpu as pltpu

_M, _K, _D = 6, 1024, 64
_P = 1024


def _vq_encode_kernel(x_ref, cb_ref, out_ref):
    x = x_ref[0]
    cb = cb_ref[0]
    inter = jax.lax.dot_general(
        x, cb, (((0,), (1,)), ((), ())), preferred_element_type=jnp.float32
    )
    q2 = jnp.sum(x * x, axis=0)[:, None]
    c2 = jnp.sum(cb * cb, axis=1)[None, :]
    dist = -(q2 + c2 - 2.0 * inter)
    best = jnp.max(dist, axis=1, keepdims=True)
    iota_k = jax.lax.broadcasted_iota(jnp.int32, (_P, _K), 1)
    idx = jnp.min(jnp.where(dist == best, iota_k, _K), axis=1, keepdims=True)
    out_ref[0, 0] = idx.reshape(8, 128)


def kernel(latent, codebook):
    n, c, h, w = latent.shape
    p = h * w
    lat = latent.reshape(n, c, p)
    out = pl.pallas_call(
        _vq_encode_kernel,
        grid=(_M, n),
        in_specs=[
            pl.BlockSpec((1, _D, p), lambda m, i: (i, m, 0)),
            pl.BlockSpec((1, _K, _D), lambda m, i: (m, 0, 0)),
        ],
        out_specs=pl.BlockSpec((1, 1, 8, 128), lambda m, i: (m, i, 0, 0)),
        out_shape=jax.ShapeDtypeStruct((_M, n, 8, 128), jnp.int32),
    )(lat, codebook)
    return out.reshape(_M, n, h, w).transpose(1, 2, 3, 0)


# speedup vs baseline: 1.9313x; 1.3313x over previous
import jax
import jax.numpy as jnp
from jax.experimental import pallas as pl
from jax.experimental.pallas import tpu as pltpu

_M, _K, _D = 6, 1024, 64
_P = 1024


def _vq_encode_kernel(x_ref, cb_ref, c2_ref, out_ref):
    x = x_ref[0]
    cb = cb_ref[0]
    inter2 = jax.lax.dot_general(
        x + x, cb, (((0,), (1,)), ((), ())), preferred_element_type=jnp.float32
    )
    q2 = jnp.sum(x * x, axis=0)[:, None]
    dist = inter2 - (q2 + c2_ref[0])
    idx = jnp.argmax(dist, axis=1).astype(jnp.int32)
    out_ref[0, 0] = idx.reshape(8, 128)


def kernel(latent, codebook):
    n, c, h, w = latent.shape
    p = h * w
    lat = latent.reshape(n, c, p)
    c2 = jnp.sum(codebook**2, axis=-1)[:, None, :]
    out = pl.pallas_call(
        _vq_encode_kernel,
        grid=(_M, n),
        in_specs=[
            pl.BlockSpec((1, _D, p), lambda m, i: (i, m, 0)),
            pl.BlockSpec((1, _K, _D), lambda m, i: (m, 0, 0)),
            pl.BlockSpec((1, 1, _K), lambda m, i: (m, 0, 0)),
        ],
        out_specs=pl.BlockSpec((1, 1, 8, 128), lambda m, i: (m, i, 0, 0)),
        out_shape=jax.ShapeDtypeStruct((_M, n, 8, 128), jnp.int32),
    )(lat, codebook, c2)
    return out.reshape(_M, n, h, w).transpose(1, 2, 3, 0)


# speedup vs baseline: 2.6767x; 1.3859x over previous
import jax
import jax.numpy as jnp
from jax.experimental import pallas as pl
from jax.experimental.pallas import tpu as pltpu

_M, _K, _D = 6, 1024, 64
_P = 1024


def _vq_encode_kernel(x_ref, cb_ref, c2_ref, out_ref):
    x = x_ref[0]
    cb = cb_ref[0]
    inter2 = jax.lax.dot_general(
        cb, x + x, (((1,), (0,)), ((), ())), preferred_element_type=jnp.float32
    )
    q2 = jnp.sum(x * x, axis=0)[None, :]
    dist = inter2 - (q2 + c2_ref[0])
    idx = jnp.argmax(dist, axis=0).astype(jnp.int32)
    out_ref[0, 0] = idx.reshape(8, 128)


def kernel(latent, codebook):
    n, c, h, w = latent.shape
    p = h * w
    lat = latent.reshape(n, c, p)
    c2 = jnp.sum(codebook**2, axis=-1)[:, :, None]
    out = pl.pallas_call(
        _vq_encode_kernel,
        grid=(_M, n),
        in_specs=[
            pl.BlockSpec((1, _D, p), lambda m, i: (i, m, 0)),
            pl.BlockSpec((1, _K, _D), lambda m, i: (m, 0, 0)),
            pl.BlockSpec((1, _K, 1), lambda m, i: (m, 0, 0)),
        ],
        out_specs=pl.BlockSpec((1, 1, 8, 128), lambda m, i: (m, i, 0, 0)),
        out_shape=jax.ShapeDtypeStruct((_M, n, 8, 128), jnp.int32),
    )(lat, codebook, c2)
    return out.reshape(_M, n, h, w).transpose(1, 2, 3, 0)
